# Initial kernel scaffold; baseline (speedup 1.0000x reference)
import functools

import jax
import jax.numpy as jnp
from jax import lax
from jax.experimental import pallas as pl
from jax.experimental.pallas import tpu as pltpu
from jax.experimental.pallas import tpu_sc as plsc

NC = 2
NS = 16
NW = NC * NS
K = 128


def _mesh():
    return plsc.VectorSubcoreMesh(core_axis_name="c", subcore_axis_name="s")


def _sc_degree(dst_r, n_pad, cpt):
    rpt = n_pad // NS

    @functools.partial(
        pl.kernel,
        out_type=jax.ShapeDtypeStruct((NC, n_pad), jnp.float32),
        mesh=_mesh(),
        scratch_types=[
            pltpu.VMEM((cpt, K), jnp.int32),
            pltpu.VMEM((K,), jnp.float32),
            pltpu.VMEM((rpt,), jnp.float32),
            pltpu.VMEM_SHARED((n_pad,), jnp.float32),
        ],
    )
    def k(dst_hbm, out_hbm, dst_v, ones_v, zeros_v, acc_sh):
        c = lax.axis_index("c")
        s = lax.axis_index("s")
        wid = c * NS + s

        def fill_ones(i, _):
            ones_v[pl.ds(i * 16, 16)] = jnp.full((16,), 1.0, jnp.float32)
            return 0

        lax.fori_loop(0, K // 16, fill_ones, 0)

        def fill_zeros(i, _):
            zeros_v[pl.ds(i * 16, 16)] = jnp.zeros((16,), jnp.float32)
            return 0

        lax.fori_loop(0, rpt // 16, fill_zeros, 0)

        pltpu.sync_copy(zeros_v, acc_sh.at[pl.ds(s * rpt, rpt)])
        plsc.subcore_barrier()

        pltpu.sync_copy(dst_hbm.at[wid], dst_v)

        def chunk(j, _):
            pltpu.sync_copy(ones_v, acc_sh.at[dst_v.at[j]], add=True)
            return 0

        lax.fori_loop(0, cpt, chunk, 0)
        plsc.subcore_barrier()
        pltpu.sync_copy(acc_sh.at[pl.ds(s * rpt, rpt)],
                        out_hbm.at[c, pl.ds(s * rpt, rpt)])

    return k(dst_r)


def _sc_agg(hp, src_r, dst_r, n_pad, cpt, h):
    rpt = n_pad // NS
    zr = 64

    @functools.partial(
        pl.kernel,
        out_type=jax.ShapeDtypeStruct((NC, n_pad, h), jnp.float32),
        mesh=_mesh(),
        scratch_types=[
            pltpu.VMEM((cpt, K), jnp.int32),
            pltpu.VMEM((cpt, K), jnp.int32),
            pltpu.VMEM((K, h), jnp.float32),
            pltpu.VMEM((zr, h), jnp.float32),
            pltpu.SemaphoreType.DMA,
            pltpu.VMEM_SHARED((n_pad, h), jnp.float32),
        ],
    )
    def k(hp_hbm, src_hbm, dst_hbm, out_hbm,
          src_v, dst_v, rows_v, zer_v, sem, acc_sh):
        c = lax.axis_index("c")
        s = lax.axis_index("s")
        wid = c * NS + s

        hvecs = h // 16

        def zf(i, _):
            zer_v[i // hvecs, pl.ds((i % hvecs) * 16, 16)] = (
                jnp.zeros((16,), jnp.float32))
            return 0

        lax.fori_loop(0, zr * hvecs, zf, 0)

        def zc(t, _):
            pltpu.sync_copy(zer_v, acc_sh.at[pl.ds(s * rpt + t * zr, zr)])
            return 0

        lax.fori_loop(0, rpt // zr, zc, 0)
        plsc.subcore_barrier()

        pltpu.sync_copy(src_hbm.at[wid], src_v)
        pltpu.sync_copy(dst_hbm.at[wid], dst_v)

        def chunk(j, _):
            pltpu.async_copy(hp_hbm.at[src_v.at[j]], rows_v, sem).wait()
            pltpu.sync_copy(rows_v, acc_sh.at[dst_v.at[j]], add=True)
            return 0

        lax.fori_loop(0, cpt, chunk, 0)
        plsc.subcore_barrier()
        pltpu.sync_copy(acc_sh.at[pl.ds(s * rpt, rpt)],
                        out_hbm.at[c, pl.ds(s * rpt, rpt)])

    return k(hp, src_r, dst_r)


def _tc_pre(featp, W1, degp, n_pad, blk):
    f = featp.shape[1]
    h1 = W1.shape[1]

    def body(feat_ref, w_ref, degp_ref, hp_ref, dinv_ref):
        deg = degp_ref[0, :] + degp_ref[1, :] + 1.0
        dinv = lax.rsqrt(deg)
        dinv_ref[:] = dinv
        hm = jnp.dot(feat_ref[:, :], w_ref[:, :],
                     preferred_element_type=jnp.float32)
        hp_ref[:, :] = hm * dinv[:, None]

    return pl.pallas_call(
        body,
        grid=(n_pad // blk,),
        in_specs=[
            pl.BlockSpec((blk, f), lambda i: (i, 0)),
            pl.BlockSpec((f, h1), lambda i: (0, 0)),
            pl.BlockSpec((NC, blk), lambda i: (0, i)),
        ],
        out_specs=[
            pl.BlockSpec((blk, h1), lambda i: (i, 0)),
            pl.BlockSpec((blk,), lambda i: (i,)),
        ],
        out_shape=[
            jax.ShapeDtypeStruct((n_pad, h1), jnp.float32),
            jax.ShapeDtypeStruct((n_pad,), jnp.float32),
        ],
    )(featp, W1, degp)


def _tc_mid(aggp, hp1, dinv, b1, W2, n_valid, n_pad, blk):
    h1 = hp1.shape[1]
    h2 = W2.shape[1]

    def body(aggp_ref, hp_ref, dinv_ref, b_ref, w_ref, out_ref):
        i = pl.program_id(0)
        agg = aggp_ref[0, :, :] + aggp_ref[1, :, :]
        dinv = dinv_ref[:]
        t = (agg + hp_ref[:, :]) * dinv[:, None] + b_ref[0, :]
        t = jnp.maximum(t, 0.0)
        o = jnp.dot(t, w_ref[:, :], preferred_element_type=jnp.float32)
        o = o * dinv[:, None]
        row = i * blk + lax.broadcasted_iota(jnp.int32, (blk, 1), 0)
        out_ref[:, :] = jnp.where(row < n_valid, o, 0.0)

    return pl.pallas_call(
        body,
        grid=(n_pad // blk,),
        in_specs=[
            pl.BlockSpec((NC, blk, h1), lambda i: (0, i, 0)),
            pl.BlockSpec((blk, h1), lambda i: (i, 0)),
            pl.BlockSpec((blk,), lambda i: (i,)),
            pl.BlockSpec((1, h1), lambda i: (0, 0)),
            pl.BlockSpec((h1, h2), lambda i: (0, 0)),
        ],
        out_specs=pl.BlockSpec((blk, h2), lambda i: (i, 0)),
        out_shape=jax.ShapeDtypeStruct((n_pad, h2), jnp.float32),
    )(aggp, hp1, dinv, b1, W2)


def _tc_head(aggp, hp2, dinv, b2, Wf, bf, n_valid, n_pad, blk):
    h2 = hp2.shape[1]
    c_dim = Wf.shape[1]

    def body(aggp_ref, hp_ref, dinv_ref, b_ref, wf_ref, bf_ref, out_ref):
        i = pl.program_id(0)
        agg = aggp_ref[0, :, :] + aggp_ref[1, :, :]
        dinv = dinv_ref[:]
        t = (agg + hp_ref[:, :]) * dinv[:, None] + b_ref[0, :]
        t = jnp.maximum(t, 0.0)
        row = i * blk + lax.broadcasted_iota(jnp.int32, (blk, 1), 0)
        t = jnp.where(row < n_valid, t, 0.0)
        logits = jnp.dot(t, wf_ref[:, :],
                         preferred_element_type=jnp.float32) + bf_ref[0, :]
        m = jnp.max(logits, axis=1, keepdims=True)
        lse = jnp.log(jnp.sum(jnp.exp(logits - m), axis=1, keepdims=True)) + m
        out_ref[:, :] = logits - lse

    return pl.pallas_call(
        body,
        grid=(n_pad // blk,),
        in_specs=[
            pl.BlockSpec((NC, blk, h2), lambda i: (0, i, 0)),
            pl.BlockSpec((blk, h2), lambda i: (i, 0)),
            pl.BlockSpec((blk,), lambda i: (i,)),
            pl.BlockSpec((1, h2), lambda i: (0, 0)),
            pl.BlockSpec((h2, c_dim), lambda i: (0, 0)),
            pl.BlockSpec((1, c_dim), lambda i: (0, 0)),
        ],
        out_specs=pl.BlockSpec((blk, c_dim), lambda i: (i, 0)),
        out_shape=jax.ShapeDtypeStruct((n_pad, c_dim), jnp.float32),
    )(aggp, hp2, dinv, b2, Wf, bf)


def kernel(feature, edge_index, W1, b1, W2, b2, Wf, bf):
    n, _ = feature.shape
    e = edge_index.shape[1]
    blk = 1024
    n_pad = -(-n // blk) * blk
    epw = NW * K
    e_pad = -(-e // epw) * epw
    cpt = e_pad // epw

    pad_node = n_pad - 1
    src = jnp.concatenate(
        [edge_index[0], jnp.full((e_pad - e,), pad_node, jnp.int32)])
    dst = jnp.concatenate(
        [edge_index[1], jnp.full((e_pad - e,), pad_node, jnp.int32)])
    src_r = src.reshape(NW, cpt, K)
    dst_r = dst.reshape(NW, cpt, K)
    featp = jnp.pad(feature, ((0, n_pad - n), (0, 0)))

    degp = _sc_degree(dst_r, n_pad, cpt)
    hp1, dinv = _tc_pre(featp, W1, degp, n_pad, blk)
    aggp1 = _sc_agg(hp1, src_r, dst_r, n_pad, cpt, W1.shape[1])
    hp2 = _tc_mid(aggp1, hp1, dinv, b1.reshape(1, -1), W2, n, n_pad, blk)
    aggp2 = _sc_agg(hp2, src_r, dst_r, n_pad, cpt, W2.shape[1])
    logp = _tc_head(aggp2, hp2, dinv, b2.reshape(1, -1), Wf,
                    bf.reshape(1, -1), n, n_pad, blk)
    return logp[:n]


# speedup vs baseline: 22.9251x; 22.9251x over previous
import functools

import jax
import jax.numpy as jnp
from jax import lax
from jax.experimental import pallas as pl
from jax.experimental.pallas import tpu as pltpu
from jax.experimental.pallas import tpu_sc as plsc

NC = 2
NS = 16
NW = NC * NS
K = 128


def _mesh():
    return plsc.VectorSubcoreMesh(core_axis_name="c", subcore_axis_name="s")


def _sc_degree(dst_r, n_pad, cpt):
    rpt = n_pad // NS

    @functools.partial(
        pl.kernel,
        out_type=jax.ShapeDtypeStruct((NC, n_pad), jnp.float32),
        mesh=_mesh(),
        scratch_types=[
            pltpu.VMEM((cpt, K), jnp.int32),
            pltpu.VMEM((K,), jnp.float32),
            pltpu.VMEM((rpt,), jnp.float32),
            pltpu.VMEM_SHARED((n_pad,), jnp.float32),
        ],
    )
    def k(dst_hbm, out_hbm, dst_v, ones_v, zeros_v, acc_sh):
        c = lax.axis_index("c")
        s = lax.axis_index("s")
        wid = c * NS + s

        def fill_ones(i, _):
            ones_v[pl.ds(i * 16, 16)] = jnp.full((16,), 1.0, jnp.float32)
            return 0

        lax.fori_loop(0, K // 16, fill_ones, 0)

        def fill_zeros(i, _):
            zeros_v[pl.ds(i * 16, 16)] = jnp.zeros((16,), jnp.float32)
            return 0

        lax.fori_loop(0, rpt // 16, fill_zeros, 0)

        pltpu.sync_copy(zeros_v, acc_sh.at[pl.ds(s * rpt, rpt)])
        plsc.subcore_barrier()

        pltpu.sync_copy(dst_hbm.at[wid], dst_v)

        def chunk(j, _):
            pltpu.sync_copy(ones_v, acc_sh.at[dst_v.at[j]], add=True)
            return 0

        lax.fori_loop(0, cpt, chunk, 0)
        plsc.subcore_barrier()
        pltpu.sync_copy(acc_sh.at[pl.ds(s * rpt, rpt)],
                        out_hbm.at[c, pl.ds(s * rpt, rpt)])

    return k(dst_r)


def _sc_agg(hp, src_r, dst_r, n_pad, cpt, h):
    rpt = n_pad // NS
    zr = 64

    @functools.partial(
        pl.kernel,
        out_type=jax.ShapeDtypeStruct((NC, n_pad, h), jnp.float32),
        mesh=_mesh(),
        scratch_types=[
            pltpu.VMEM((cpt, K), jnp.int32),
            pltpu.VMEM((cpt, K), jnp.int32),
            pltpu.VMEM((K, h), jnp.float32),
            pltpu.VMEM((zr, h), jnp.float32),
            pltpu.SemaphoreType.DMA,
            pltpu.VMEM_SHARED((n_pad, h), jnp.float32),
        ],
        compiler_params=pltpu.CompilerParams(use_tc_tiling_on_sc=False),
    )
    def k(hp_hbm, src_hbm, dst_hbm, out_hbm,
          src_v, dst_v, rows_v, zer_v, sem, acc_sh):
        c = lax.axis_index("c")
        s = lax.axis_index("s")
        wid = c * NS + s

        hvecs = h // 16

        def zf(i, _):
            zer_v[i // hvecs, pl.ds((i % hvecs) * 16, 16)] = (
                jnp.zeros((16,), jnp.float32))
            return 0

        lax.fori_loop(0, zr * hvecs, zf, 0)

        def zc(t, _):
            pltpu.sync_copy(zer_v, acc_sh.at[pl.ds(s * rpt + t * zr, zr)])
            return 0

        lax.fori_loop(0, rpt // zr, zc, 0)
        plsc.subcore_barrier()

        pltpu.sync_copy(src_hbm.at[wid], src_v)
        pltpu.sync_copy(dst_hbm.at[wid], dst_v)

        def chunk(j, _):
            pltpu.async_copy(hp_hbm.at[src_v.at[j]], rows_v, sem).wait()
            pltpu.sync_copy(rows_v, acc_sh.at[dst_v.at[j]], add=True)
            return 0

        lax.fori_loop(0, cpt, chunk, 0)
        plsc.subcore_barrier()
        pltpu.sync_copy(acc_sh.at[pl.ds(s * rpt, rpt)],
                        out_hbm.at[c, pl.ds(s * rpt, rpt)])

    return k(hp, src_r, dst_r)


def _tc_pre(featp, W1, degp, n_pad, blk):
    f = featp.shape[1]
    h1 = W1.shape[1]

    def body(feat_ref, w_ref, degp_ref, hp_ref, dinv_ref):
        deg = degp_ref[0, :] + degp_ref[1, :] + 1.0
        dinv = lax.rsqrt(deg)
        dinv_ref[:] = dinv
        hm = jnp.dot(feat_ref[:, :], w_ref[:, :],
                     preferred_element_type=jnp.float32)
        hp_ref[:, :] = hm * dinv[:, None]

    return pl.pallas_call(
        body,
        grid=(n_pad // blk,),
        in_specs=[
            pl.BlockSpec((blk, f), lambda i: (i, 0)),
            pl.BlockSpec((f, h1), lambda i: (0, 0)),
            pl.BlockSpec((NC, blk), lambda i: (0, i)),
        ],
        out_specs=[
            pl.BlockSpec((blk, h1), lambda i: (i, 0)),
            pl.BlockSpec((blk,), lambda i: (i,)),
        ],
        out_shape=[
            jax.ShapeDtypeStruct((n_pad, h1), jnp.float32),
            jax.ShapeDtypeStruct((n_pad,), jnp.float32),
        ],
    )(featp, W1, degp)


def _tc_mid(aggp, hp1, dinv, b1, W2, n_valid, n_pad, blk):
    h1 = hp1.shape[1]
    h2 = W2.shape[1]

    def body(aggp_ref, hp_ref, dinv_ref, b_ref, w_ref, out_ref):
        i = pl.program_id(0)
        agg = aggp_ref[0, :, :] + aggp_ref[1, :, :]
        dinv = dinv_ref[:]
        t = (agg + hp_ref[:, :]) * dinv[:, None] + b_ref[0, :]
        t = jnp.maximum(t, 0.0)
        o = jnp.dot(t, w_ref[:, :], preferred_element_type=jnp.float32)
        o = o * dinv[:, None]
        row = i * blk + lax.broadcasted_iota(jnp.int32, (blk, 1), 0)
        out_ref[:, :] = jnp.where(row < n_valid, o, 0.0)

    return pl.pallas_call(
        body,
        grid=(n_pad // blk,),
        in_specs=[
            pl.BlockSpec((NC, blk, h1), lambda i: (0, i, 0)),
            pl.BlockSpec((blk, h1), lambda i: (i, 0)),
            pl.BlockSpec((blk,), lambda i: (i,)),
            pl.BlockSpec((1, h1), lambda i: (0, 0)),
            pl.BlockSpec((h1, h2), lambda i: (0, 0)),
        ],
        out_specs=pl.BlockSpec((blk, h2), lambda i: (i, 0)),
        out_shape=jax.ShapeDtypeStruct((n_pad, h2), jnp.float32),
    )(aggp, hp1, dinv, b1, W2)


def _tc_head(aggp, hp2, dinv, b2, Wf, bf, n_valid, n_pad, blk):
    h2 = hp2.shape[1]
    c_dim = Wf.shape[1]

    def body(aggp_ref, hp_ref, dinv_ref, b_ref, wf_ref, bf_ref, out_ref):
        i = pl.program_id(0)
        agg = aggp_ref[0, :, :] + aggp_ref[1, :, :]
        dinv = dinv_ref[:]
        t = (agg + hp_ref[:, :]) * dinv[:, None] + b_ref[0, :]
        t = jnp.maximum(t, 0.0)
        row = i * blk + lax.broadcasted_iota(jnp.int32, (blk, 1), 0)
        t = jnp.where(row < n_valid, t, 0.0)
        logits = jnp.dot(t, wf_ref[:, :],
                         preferred_element_type=jnp.float32) + bf_ref[0, :]
        m = jnp.max(logits, axis=1, keepdims=True)
        lse = jnp.log(jnp.sum(jnp.exp(logits - m), axis=1, keepdims=True)) + m
        out_ref[:, :] = logits - lse

    return pl.pallas_call(
        body,
        grid=(n_pad // blk,),
        in_specs=[
            pl.BlockSpec((NC, blk, h2), lambda i: (0, i, 0)),
            pl.BlockSpec((blk, h2), lambda i: (i, 0)),
            pl.BlockSpec((blk,), lambda i: (i,)),
            pl.BlockSpec((1, h2), lambda i: (0, 0)),
            pl.BlockSpec((h2, c_dim), lambda i: (0, 0)),
            pl.BlockSpec((1, c_dim), lambda i: (0, 0)),
        ],
        out_specs=pl.BlockSpec((blk, c_dim), lambda i: (i, 0)),
        out_shape=jax.ShapeDtypeStruct((n_pad, c_dim), jnp.float32),
    )(aggp, hp2, dinv, b2, Wf, bf)


def kernel(feature, edge_index, W1, b1, W2, b2, Wf, bf):
    n, _ = feature.shape
    e = edge_index.shape[1]
    blk = 1024
    n_pad = -(-n // blk) * blk
    epw = NW * K
    e_pad = -(-e // epw) * epw
    cpt = e_pad // epw

    pad_node = n_pad - 1
    src = jnp.concatenate(
        [edge_index[0], jnp.full((e_pad - e,), pad_node, jnp.int32)])
    dst = jnp.concatenate(
        [edge_index[1], jnp.full((e_pad - e,), pad_node, jnp.int32)])
    src_r = src.reshape(NW, cpt, K)
    dst_r = dst.reshape(NW, cpt, K)
    featp = jnp.pad(feature, ((0, n_pad - n), (0, 0)))

    degp = _sc_degree(dst_r, n_pad, cpt)
    hp1, dinv = _tc_pre(featp, W1, degp, n_pad, blk)
    aggp1 = _sc_agg(hp1, src_r, dst_r, n_pad, cpt, W1.shape[1])
    hp2 = _tc_mid(aggp1, hp1, dinv, b1.reshape(1, -1), W2, n, n_pad, blk)
    aggp2 = _sc_agg(hp2, src_r, dst_r, n_pad, cpt, W2.shape[1])
    logp = _tc_head(aggp2, hp2, dinv, b2.reshape(1, -1), Wf,
                    bf.reshape(1, -1), n, n_pad, blk)
    return logp[:n]


# speedup vs baseline: 25.5242x; 1.1134x over previous
import functools

import jax
import jax.numpy as jnp
from jax import lax
from jax.experimental import pallas as pl
from jax.experimental.pallas import tpu as pltpu
from jax.experimental.pallas import tpu_sc as plsc

NC = 2
NS = 16
NW = NC * NS
K = 128


def _mesh():
    return plsc.VectorSubcoreMesh(core_axis_name="c", subcore_axis_name="s")


def _sc_degree(dst_r, n_pad, cpt):
    rpt = n_pad // NS

    @functools.partial(
        pl.kernel,
        out_type=jax.ShapeDtypeStruct((NC, n_pad), jnp.float32),
        mesh=_mesh(),
        scratch_types=[
            pltpu.VMEM((cpt, K), jnp.int32),
            pltpu.VMEM((K,), jnp.float32),
            pltpu.VMEM((rpt,), jnp.float32),
            pltpu.VMEM_SHARED((n_pad,), jnp.float32),
        ],
    )
    def k(dst_hbm, out_hbm, dst_v, ones_v, zeros_v, acc_sh):
        c = lax.axis_index("c")
        s = lax.axis_index("s")
        wid = c * NS + s

        def fill_ones(i, _):
            ones_v[pl.ds(i * 16, 16)] = jnp.full((16,), 1.0, jnp.float32)
            return 0

        lax.fori_loop(0, K // 16, fill_ones, 0)

        def fill_zeros(i, _):
            zeros_v[pl.ds(i * 16, 16)] = jnp.zeros((16,), jnp.float32)
            return 0

        lax.fori_loop(0, rpt // 16, fill_zeros, 0)

        pltpu.sync_copy(zeros_v, acc_sh.at[pl.ds(s * rpt, rpt)])
        plsc.subcore_barrier()

        pltpu.sync_copy(dst_hbm.at[wid], dst_v)

        def chunk(j, _):
            pltpu.sync_copy(ones_v, acc_sh.at[dst_v.at[j]], add=True)
            return 0

        lax.fori_loop(0, cpt, chunk, 0)
        plsc.subcore_barrier()
        pltpu.sync_copy(acc_sh.at[pl.ds(s * rpt, rpt)],
                        out_hbm.at[c, pl.ds(s * rpt, rpt)])

    return k(dst_r)


def _sc_agg(hp, src_r, dst_r, n_pad, cpt, h):
    rpt = n_pad // NS
    zr = 64

    @functools.partial(
        pl.kernel,
        out_type=jax.ShapeDtypeStruct((NC, n_pad, h), jnp.float32),
        mesh=_mesh(),
        scratch_types=[
            pltpu.VMEM((cpt, K), jnp.int32),
            pltpu.VMEM((cpt, K), jnp.int32),
            pltpu.VMEM((2, K, h), jnp.float32),
            pltpu.VMEM((zr, h), jnp.float32),
            pltpu.SemaphoreType.DMA,
            pltpu.VMEM_SHARED((n_pad, h), jnp.float32),
        ],
        compiler_params=pltpu.CompilerParams(use_tc_tiling_on_sc=False),
    )
    def k(hp_hbm, src_hbm, dst_hbm, out_hbm,
          src_v, dst_v, rows_v, zer_v, sem, acc_sh):
        c = lax.axis_index("c")
        s = lax.axis_index("s")
        wid = c * NS + s

        hvecs = h // 16

        def zf(i, _):
            zer_v[i // hvecs, pl.ds((i % hvecs) * 16, 16)] = (
                jnp.zeros((16,), jnp.float32))
            return 0

        lax.fori_loop(0, zr * hvecs, zf, 0)

        def zc(t, _):
            pltpu.sync_copy(zer_v, acc_sh.at[pl.ds(s * rpt + t * zr, zr)])
            return 0

        lax.fori_loop(0, rpt // zr, zc, 0)
        plsc.subcore_barrier()

        pltpu.sync_copy(src_hbm.at[wid], src_v)
        pltpu.sync_copy(dst_hbm.at[wid], dst_v)

        pltpu.async_copy(hp_hbm.at[src_v.at[0]], rows_v.at[0], sem)

        def chunk(j, _):
            buf = lax.rem(j, 2)
            pltpu.make_async_copy(
                hp_hbm.at[src_v.at[j]], rows_v.at[buf], sem).wait()

            @pl.when(j + 1 < cpt)
            def _():
                pltpu.async_copy(
                    hp_hbm.at[src_v.at[j + 1]], rows_v.at[1 - buf], sem)

            pltpu.sync_copy(rows_v.at[buf], acc_sh.at[dst_v.at[j]], add=True)
            return 0

        lax.fori_loop(0, cpt, chunk, 0)
        plsc.subcore_barrier()
        pltpu.sync_copy(acc_sh.at[pl.ds(s * rpt, rpt)],
                        out_hbm.at[c, pl.ds(s * rpt, rpt)])

    return k(hp, src_r, dst_r)


def _tc_pre(featp, W1, degp, n_pad, blk):
    f = featp.shape[1]
    h1 = W1.shape[1]

    def body(feat_ref, w_ref, degp_ref, hp_ref, dinv_ref):
        deg = degp_ref[0, :] + degp_ref[1, :] + 1.0
        dinv = lax.rsqrt(deg)
        dinv_ref[:] = dinv
        hm = jnp.dot(feat_ref[:, :], w_ref[:, :],
                     preferred_element_type=jnp.float32)
        hp_ref[:, :] = hm * dinv[:, None]

    return pl.pallas_call(
        body,
        grid=(n_pad // blk,),
        in_specs=[
            pl.BlockSpec((blk, f), lambda i: (i, 0)),
            pl.BlockSpec((f, h1), lambda i: (0, 0)),
            pl.BlockSpec((NC, blk), lambda i: (0, i)),
        ],
        out_specs=[
            pl.BlockSpec((blk, h1), lambda i: (i, 0)),
            pl.BlockSpec((blk,), lambda i: (i,)),
        ],
        out_shape=[
            jax.ShapeDtypeStruct((n_pad, h1), jnp.float32),
            jax.ShapeDtypeStruct((n_pad,), jnp.float32),
        ],
    )(featp, W1, degp)


def _tc_mid(aggp, hp1, dinv, b1, W2, n_valid, n_pad, blk):
    h1 = hp1.shape[1]
    h2 = W2.shape[1]

    def body(aggp_ref, hp_ref, dinv_ref, b_ref, w_ref, out_ref):
        i = pl.program_id(0)
        agg = aggp_ref[0, :, :] + aggp_ref[1, :, :]
        dinv = dinv_ref[:]
        t = (agg + hp_ref[:, :]) * dinv[:, None] + b_ref[0, :]
        t = jnp.maximum(t, 0.0)
        o = jnp.dot(t, w_ref[:, :], preferred_element_type=jnp.float32)
        o = o * dinv[:, None]
        row = i * blk + lax.broadcasted_iota(jnp.int32, (blk, 1), 0)
        out_ref[:, :] = jnp.where(row < n_valid, o, 0.0)

    return pl.pallas_call(
        body,
        grid=(n_pad // blk,),
        in_specs=[
            pl.BlockSpec((NC, blk, h1), lambda i: (0, i, 0)),
            pl.BlockSpec((blk, h1), lambda i: (i, 0)),
            pl.BlockSpec((blk,), lambda i: (i,)),
            pl.BlockSpec((1, h1), lambda i: (0, 0)),
            pl.BlockSpec((h1, h2), lambda i: (0, 0)),
        ],
        out_specs=pl.BlockSpec((blk, h2), lambda i: (i, 0)),
        out_shape=jax.ShapeDtypeStruct((n_pad, h2), jnp.float32),
    )(aggp, hp1, dinv, b1, W2)


def _tc_head(aggp, hp2, dinv, b2, Wf, bf, n_valid, n_pad, blk):
    h2 = hp2.shape[1]
    c_dim = Wf.shape[1]

    def body(aggp_ref, hp_ref, dinv_ref, b_ref, wf_ref, bf_ref, out_ref):
        i = pl.program_id(0)
        agg = aggp_ref[0, :, :] + aggp_ref[1, :, :]
        dinv = dinv_ref[:]
        t = (agg + hp_ref[:, :]) * dinv[:, None] + b_ref[0, :]
        t = jnp.maximum(t, 0.0)
        row = i * blk + lax.broadcasted_iota(jnp.int32, (blk, 1), 0)
        t = jnp.where(row < n_valid, t, 0.0)
        logits = jnp.dot(t, wf_ref[:, :],
                         preferred_element_type=jnp.float32) + bf_ref[0, :]
        m = jnp.max(logits, axis=1, keepdims=True)
        lse = jnp.log(jnp.sum(jnp.exp(logits - m), axis=1, keepdims=True)) + m
        out_ref[:, :] = logits - lse

    return pl.pallas_call(
        body,
        grid=(n_pad // blk,),
        in_specs=[
            pl.BlockSpec((NC, blk, h2), lambda i: (0, i, 0)),
            pl.BlockSpec((blk, h2), lambda i: (i, 0)),
            pl.BlockSpec((blk,), lambda i: (i,)),
            pl.BlockSpec((1, h2), lambda i: (0, 0)),
            pl.BlockSpec((h2, c_dim), lambda i: (0, 0)),
            pl.BlockSpec((1, c_dim), lambda i: (0, 0)),
        ],
        out_specs=pl.BlockSpec((blk, c_dim), lambda i: (i, 0)),
        out_shape=jax.ShapeDtypeStruct((n_pad, c_dim), jnp.float32),
    )(aggp, hp2, dinv, b2, Wf, bf)


def kernel(feature, edge_index, W1, b1, W2, b2, Wf, bf):
    n, _ = feature.shape
    e = edge_index.shape[1]
    blk = 1024
    n_pad = -(-n // blk) * blk
    epw = NW * K
    e_pad = -(-e // epw) * epw
    cpt = e_pad // epw

    pad_node = n_pad - 1
    src = jnp.concatenate(
        [edge_index[0], jnp.full((e_pad - e,), pad_node, jnp.int32)])
    dst = jnp.concatenate(
        [edge_index[1], jnp.full((e_pad - e,), pad_node, jnp.int32)])
    src_r = src.reshape(NW, cpt, K)
    dst_r = dst.reshape(NW, cpt, K)
    featp = jnp.pad(feature, ((0, n_pad - n), (0, 0)))

    degp = _sc_degree(dst_r, n_pad, cpt)
    hp1, dinv = _tc_pre(featp, W1, degp, n_pad, blk)
    aggp1 = _sc_agg(hp1, src_r, dst_r, n_pad, cpt, W1.shape[1])
    hp2 = _tc_mid(aggp1, hp1, dinv, b1.reshape(1, -1), W2, n, n_pad, blk)
    aggp2 = _sc_agg(hp2, src_r, dst_r, n_pad, cpt, W2.shape[1])
    logp = _tc_head(aggp2, hp2, dinv, b2.reshape(1, -1), Wf,
                    bf.reshape(1, -1), n, n_pad, blk)
    return logp[:n]


# speedup vs baseline: 30.4454x; 1.1928x over previous
import functools

import jax
import jax.numpy as jnp
from jax import lax
from jax.experimental import pallas as pl
from jax.experimental.pallas import tpu as pltpu
from jax.experimental.pallas import tpu_sc as plsc

NC = 2
NS = 16
NW = NC * NS
K = 128
NBUF = 6
SD = 3


def _mesh():
    return plsc.VectorSubcoreMesh(core_axis_name="c", subcore_axis_name="s")


def _sc_degree(dst_r, n_pad, cpt):
    rpt = n_pad // NS

    @functools.partial(
        pl.kernel,
        out_type=jax.ShapeDtypeStruct((NC, n_pad), jnp.float32),
        mesh=_mesh(),
        scratch_types=[
            pltpu.VMEM((cpt, K), jnp.int32),
            pltpu.VMEM((K,), jnp.float32),
            pltpu.VMEM((rpt,), jnp.float32),
            pltpu.VMEM_SHARED((n_pad,), jnp.float32),
        ],
    )
    def k(dst_hbm, out_hbm, dst_v, ones_v, zeros_v, acc_sh):
        c = lax.axis_index("c")
        s = lax.axis_index("s")
        wid = c * NS + s

        def fill_ones(i, _):
            ones_v[pl.ds(i * 16, 16)] = jnp.full((16,), 1.0, jnp.float32)
            return 0

        lax.fori_loop(0, K // 16, fill_ones, 0)

        def fill_zeros(i, _):
            zeros_v[pl.ds(i * 16, 16)] = jnp.zeros((16,), jnp.float32)
            return 0

        lax.fori_loop(0, rpt // 16, fill_zeros, 0)

        pltpu.sync_copy(zeros_v, acc_sh.at[pl.ds(s * rpt, rpt)])
        plsc.subcore_barrier()

        pltpu.sync_copy(dst_hbm.at[wid], dst_v)

        def chunk(j, _):
            pltpu.sync_copy(ones_v, acc_sh.at[dst_v.at[j]], add=True)
            return 0

        lax.fori_loop(0, cpt, chunk, 0)
        plsc.subcore_barrier()
        pltpu.sync_copy(acc_sh.at[pl.ds(s * rpt, rpt)],
                        out_hbm.at[c, pl.ds(s * rpt, rpt)])

    return k(dst_r)


def _sc_agg(hp, src_r, dst_r, n_pad, cpt, h):
    rpt = n_pad // NS
    zr = 64

    @functools.partial(
        pl.kernel,
        out_type=jax.ShapeDtypeStruct((NC, n_pad, h), jnp.float32),
        mesh=_mesh(),
        scratch_types=[
            pltpu.VMEM((cpt, K), jnp.int32),
            pltpu.VMEM((cpt, K), jnp.int32),
            pltpu.VMEM((NBUF, K, h), jnp.float32),
            pltpu.VMEM((zr, h), jnp.float32),
            pltpu.SemaphoreType.DMA,
            pltpu.SemaphoreType.DMA,
            pltpu.VMEM_SHARED((n_pad, h), jnp.float32),
        ],
        compiler_params=pltpu.CompilerParams(use_tc_tiling_on_sc=False),
    )
    def k(hp_hbm, src_hbm, dst_hbm, out_hbm,
          src_v, dst_v, rows_v, zer_v, sem_g, sem_s, acc_sh):
        c = lax.axis_index("c")
        s = lax.axis_index("s")
        wid = c * NS + s

        hvecs = h // 16

        def zf(i, _):
            zer_v[i // hvecs, pl.ds((i % hvecs) * 16, 16)] = (
                jnp.zeros((16,), jnp.float32))
            return 0

        lax.fori_loop(0, zr * hvecs, zf, 0)

        def zc(t, _):
            pltpu.sync_copy(zer_v, acc_sh.at[pl.ds(s * rpt + t * zr, zr)])
            return 0

        lax.fori_loop(0, rpt // zr, zc, 0)
        plsc.subcore_barrier()

        pltpu.sync_copy(src_hbm.at[wid], src_v)
        pltpu.sync_copy(dst_hbm.at[wid], dst_v)

        for b in range(NBUF - SD):
            pltpu.async_copy(hp_hbm.at[src_v.at[b]], rows_v.at[b], sem_g)

        def chunk(j, _):
            @pl.when(j >= SD)
            def _():
                pltpu.make_async_copy(
                    rows_v.at[0], acc_sh.at[dst_v.at[0]], sem_s).wait()

            g = j + NBUF - SD

            @pl.when(g < cpt)
            def _():
                pltpu.async_copy(
                    hp_hbm.at[src_v.at[g]], rows_v.at[lax.rem(g, NBUF)],
                    sem_g)

            buf = lax.rem(j, NBUF)
            pltpu.make_async_copy(
                hp_hbm.at[src_v.at[j]], rows_v.at[buf], sem_g).wait()
            pltpu.async_copy(
                rows_v.at[buf], acc_sh.at[dst_v.at[j]], sem_s, add=True)
            return 0

        lax.fori_loop(0, cpt, chunk, 0)
        for _ in range(SD):
            pltpu.make_async_copy(
                rows_v.at[0], acc_sh.at[dst_v.at[0]], sem_s).wait()
        plsc.subcore_barrier()
        pltpu.sync_copy(acc_sh.at[pl.ds(s * rpt, rpt)],
                        out_hbm.at[c, pl.ds(s * rpt, rpt)])

    return k(hp, src_r, dst_r)


def _tc_pre(featp, W1, degp, n_pad, blk):
    f = featp.shape[1]
    h1 = W1.shape[1]

    def body(feat_ref, w_ref, degp_ref, hp_ref, dinv_ref):
        deg = degp_ref[0, :] + degp_ref[1, :] + 1.0
        dinv = lax.rsqrt(deg)
        dinv_ref[:] = dinv
        hm = jnp.dot(feat_ref[:, :], w_ref[:, :],
                     preferred_element_type=jnp.float32)
        hp_ref[:, :] = hm * dinv[:, None]

    return pl.pallas_call(
        body,
        grid=(n_pad // blk,),
        in_specs=[
            pl.BlockSpec((blk, f), lambda i: (i, 0)),
            pl.BlockSpec((f, h1), lambda i: (0, 0)),
            pl.BlockSpec((NC, blk), lambda i: (0, i)),
        ],
        out_specs=[
            pl.BlockSpec((blk, h1), lambda i: (i, 0)),
            pl.BlockSpec((blk,), lambda i: (i,)),
        ],
        out_shape=[
            jax.ShapeDtypeStruct((n_pad, h1), jnp.float32),
            jax.ShapeDtypeStruct((n_pad,), jnp.float32),
        ],
    )(featp, W1, degp)


def _tc_mid(aggp, hp1, dinv, b1, W2, n_valid, n_pad, blk):
    h1 = hp1.shape[1]
    h2 = W2.shape[1]

    def body(aggp_ref, hp_ref, dinv_ref, b_ref, w_ref, out_ref):
        i = pl.program_id(0)
        agg = aggp_ref[0, :, :] + aggp_ref[1, :, :]
        dinv = dinv_ref[:]
        t = (agg + hp_ref[:, :]) * dinv[:, None] + b_ref[0, :]
        t = jnp.maximum(t, 0.0)
        o = jnp.dot(t, w_ref[:, :], preferred_element_type=jnp.float32)
        o = o * dinv[:, None]
        row = i * blk + lax.broadcasted_iota(jnp.int32, (blk, 1), 0)
        out_ref[:, :] = jnp.where(row < n_valid, o, 0.0)

    return pl.pallas_call(
        body,
        grid=(n_pad // blk,),
        in_specs=[
            pl.BlockSpec((NC, blk, h1), lambda i: (0, i, 0)),
            pl.BlockSpec((blk, h1), lambda i: (i, 0)),
            pl.BlockSpec((blk,), lambda i: (i,)),
            pl.BlockSpec((1, h1), lambda i: (0, 0)),
            pl.BlockSpec((h1, h2), lambda i: (0, 0)),
        ],
        out_specs=pl.BlockSpec((blk, h2), lambda i: (i, 0)),
        out_shape=jax.ShapeDtypeStruct((n_pad, h2), jnp.float32),
    )(aggp, hp1, dinv, b1, W2)


def _tc_head(aggp, hp2, dinv, b2, Wf, bf, n_valid, n_pad, blk):
    h2 = hp2.shape[1]
    c_dim = Wf.shape[1]

    def body(aggp_ref, hp_ref, dinv_ref, b_ref, wf_ref, bf_ref, out_ref):
        i = pl.program_id(0)
        agg = aggp_ref[0, :, :] + aggp_ref[1, :, :]
        dinv = dinv_ref[:]
        t = (agg + hp_ref[:, :]) * dinv[:, None] + b_ref[0, :]
        t = jnp.maximum(t, 0.0)
        row = i * blk + lax.broadcasted_iota(jnp.int32, (blk, 1), 0)
        t = jnp.where(row < n_valid, t, 0.0)
        logits = jnp.dot(t, wf_ref[:, :],
                         preferred_element_type=jnp.float32) + bf_ref[0, :]
        m = jnp.max(logits, axis=1, keepdims=True)
        lse = jnp.log(jnp.sum(jnp.exp(logits - m), axis=1, keepdims=True)) + m
        out_ref[:, :] = logits - lse

    return pl.pallas_call(
        body,
        grid=(n_pad // blk,),
        in_specs=[
            pl.BlockSpec((NC, blk, h2), lambda i: (0, i, 0)),
            pl.BlockSpec((blk, h2), lambda i: (i, 0)),
            pl.BlockSpec((blk,), lambda i: (i,)),
            pl.BlockSpec((1, h2), lambda i: (0, 0)),
            pl.BlockSpec((h2, c_dim), lambda i: (0, 0)),
            pl.BlockSpec((1, c_dim), lambda i: (0, 0)),
        ],
        out_specs=pl.BlockSpec((blk, c_dim), lambda i: (i, 0)),
        out_shape=jax.ShapeDtypeStruct((n_pad, c_dim), jnp.float32),
    )(aggp, hp2, dinv, b2, Wf, bf)


def kernel(feature, edge_index, W1, b1, W2, b2, Wf, bf):
    n, _ = feature.shape
    e = edge_index.shape[1]
    blk = 1024
    n_pad = -(-n // blk) * blk
    epw = NW * K
    e_pad = -(-e // epw) * epw
    cpt = e_pad // epw

    pad_node = n_pad - 1
    src = jnp.concatenate(
        [edge_index[0], jnp.full((e_pad - e,), pad_node, jnp.int32)])
    dst = jnp.concatenate(
        [edge_index[1], jnp.full((e_pad - e,), pad_node, jnp.int32)])
    src_r = src.reshape(NW, cpt, K)
    dst_r = dst.reshape(NW, cpt, K)
    featp = jnp.pad(feature, ((0, n_pad - n), (0, 0)))

    degp = _sc_degree(dst_r, n_pad, cpt)
    hp1, dinv = _tc_pre(featp, W1, degp, n_pad, blk)
    aggp1 = _sc_agg(hp1, src_r, dst_r, n_pad, cpt, W1.shape[1])
    hp2 = _tc_mid(aggp1, hp1, dinv, b1.reshape(1, -1), W2, n, n_pad, blk)
    aggp2 = _sc_agg(hp2, src_r, dst_r, n_pad, cpt, W2.shape[1])
    logp = _tc_head(aggp2, hp2, dinv, b2.reshape(1, -1), Wf,
                    bf.reshape(1, -1), n, n_pad, blk)
    return logp[:n]


# speedup vs baseline: 30.5410x; 1.0031x over previous
import functools

import jax
import jax.numpy as jnp
from jax import lax
from jax.experimental import pallas as pl
from jax.experimental.pallas import tpu as pltpu
from jax.experimental.pallas import tpu_sc as plsc

NC = 2
NS = 16
NW = NC * NS
K = 128
NBUF = 8
SD = 4


def _mesh():
    return plsc.VectorSubcoreMesh(core_axis_name="c", subcore_axis_name="s")


def _sc_degree(dst_r, n_pad, cpt):
    rpt = n_pad // NS

    @functools.partial(
        pl.kernel,
        out_type=jax.ShapeDtypeStruct((NC, n_pad), jnp.float32),
        mesh=_mesh(),
        scratch_types=[
            pltpu.VMEM((cpt, K), jnp.int32),
            pltpu.VMEM((K,), jnp.float32),
            pltpu.VMEM((rpt,), jnp.float32),
            pltpu.VMEM_SHARED((n_pad,), jnp.float32),
        ],
    )
    def k(dst_hbm, out_hbm, dst_v, ones_v, zeros_v, acc_sh):
        c = lax.axis_index("c")
        s = lax.axis_index("s")
        wid = c * NS + s

        def fill_ones(i, _):
            ones_v[pl.ds(i * 16, 16)] = jnp.full((16,), 1.0, jnp.float32)
            return 0

        lax.fori_loop(0, K // 16, fill_ones, 0)

        def fill_zeros(i, _):
            zeros_v[pl.ds(i * 16, 16)] = jnp.zeros((16,), jnp.float32)
            return 0

        lax.fori_loop(0, rpt // 16, fill_zeros, 0)

        pltpu.sync_copy(zeros_v, acc_sh.at[pl.ds(s * rpt, rpt)])
        plsc.subcore_barrier()

        pltpu.sync_copy(dst_hbm.at[wid], dst_v)

        def chunk(j, _):
            pltpu.sync_copy(ones_v, acc_sh.at[dst_v.at[j]], add=True)
            return 0

        lax.fori_loop(0, cpt, chunk, 0)
        plsc.subcore_barrier()
        pltpu.sync_copy(acc_sh.at[pl.ds(s * rpt, rpt)],
                        out_hbm.at[c, pl.ds(s * rpt, rpt)])

    return k(dst_r)


def _sc_agg(hp, src_r, dst_r, n_pad, cpt, h):
    rpt = n_pad // NS
    zr = 64

    @functools.partial(
        pl.kernel,
        out_type=jax.ShapeDtypeStruct((NC, n_pad, h), jnp.float32),
        mesh=_mesh(),
        scratch_types=[
            pltpu.VMEM((cpt, K), jnp.int32),
            pltpu.VMEM((cpt, K), jnp.int32),
            pltpu.VMEM((NBUF, K, h), jnp.float32),
            pltpu.VMEM((zr, h), jnp.float32),
            pltpu.SemaphoreType.DMA,
            pltpu.SemaphoreType.DMA,
            pltpu.VMEM_SHARED((n_pad, h), jnp.float32),
        ],
        compiler_params=pltpu.CompilerParams(use_tc_tiling_on_sc=False),
    )
    def k(hp_hbm, src_hbm, dst_hbm, out_hbm,
          src_v, dst_v, rows_v, zer_v, sem_g, sem_s, acc_sh):
        c = lax.axis_index("c")
        s = lax.axis_index("s")
        wid = c * NS + s

        hvecs = h // 16

        def zf(i, _):
            zer_v[i // hvecs, pl.ds((i % hvecs) * 16, 16)] = (
                jnp.zeros((16,), jnp.float32))
            return 0

        lax.fori_loop(0, zr * hvecs, zf, 0)

        def zc(t, _):
            pltpu.sync_copy(zer_v, acc_sh.at[pl.ds(s * rpt + t * zr, zr)])
            return 0

        lax.fori_loop(0, rpt // zr, zc, 0)
        plsc.subcore_barrier()

        pltpu.sync_copy(src_hbm.at[wid], src_v)
        pltpu.sync_copy(dst_hbm.at[wid], dst_v)

        for b in range(NBUF - SD):
            pltpu.async_copy(hp_hbm.at[src_v.at[b]], rows_v.at[b], sem_g)

        def chunk(j, _):
            @pl.when(j >= SD)
            def _():
                pltpu.make_async_copy(
                    rows_v.at[0], acc_sh.at[dst_v.at[0]], sem_s).wait()

            g = j + NBUF - SD

            @pl.when(g < cpt)
            def _():
                pltpu.async_copy(
                    hp_hbm.at[src_v.at[g]], rows_v.at[lax.rem(g, NBUF)],
                    sem_g)

            buf = lax.rem(j, NBUF)
            pltpu.make_async_copy(
                hp_hbm.at[src_v.at[j]], rows_v.at[buf], sem_g).wait()
            pltpu.async_copy(
                rows_v.at[buf], acc_sh.at[dst_v.at[j]], sem_s, add=True)
            return 0

        lax.fori_loop(0, cpt, chunk, 0)
        for _ in range(SD):
            pltpu.make_async_copy(
                rows_v.at[0], acc_sh.at[dst_v.at[0]], sem_s).wait()
        plsc.subcore_barrier()
        pltpu.sync_copy(acc_sh.at[pl.ds(s * rpt, rpt)],
                        out_hbm.at[c, pl.ds(s * rpt, rpt)])

    return k(hp, src_r, dst_r)


def _tc_pre(featp, W1, degp, n_pad, blk):
    f = featp.shape[1]
    h1 = W1.shape[1]

    def body(feat_ref, w_ref, degp_ref, hp_ref, dinv_ref):
        deg = degp_ref[0, :] + degp_ref[1, :] + 1.0
        dinv = lax.rsqrt(deg)
        dinv_ref[:] = dinv
        hm = jnp.dot(feat_ref[:, :], w_ref[:, :],
                     preferred_element_type=jnp.float32)
        hp_ref[:, :] = hm * dinv[:, None]

    return pl.pallas_call(
        body,
        grid=(n_pad // blk,),
        in_specs=[
            pl.BlockSpec((blk, f), lambda i: (i, 0)),
            pl.BlockSpec((f, h1), lambda i: (0, 0)),
            pl.BlockSpec((NC, blk), lambda i: (0, i)),
        ],
        out_specs=[
            pl.BlockSpec((blk, h1), lambda i: (i, 0)),
            pl.BlockSpec((blk,), lambda i: (i,)),
        ],
        out_shape=[
            jax.ShapeDtypeStruct((n_pad, h1), jnp.float32),
            jax.ShapeDtypeStruct((n_pad,), jnp.float32),
        ],
    )(featp, W1, degp)


def _tc_mid(aggp, hp1, dinv, b1, W2, n_valid, n_pad, blk):
    h1 = hp1.shape[1]
    h2 = W2.shape[1]

    def body(aggp_ref, hp_ref, dinv_ref, b_ref, w_ref, out_ref):
        i = pl.program_id(0)
        agg = aggp_ref[0, :, :] + aggp_ref[1, :, :]
        dinv = dinv_ref[:]
        t = (agg + hp_ref[:, :]) * dinv[:, None] + b_ref[0, :]
        t = jnp.maximum(t, 0.0)
        o = jnp.dot(t, w_ref[:, :], preferred_element_type=jnp.float32)
        o = o * dinv[:, None]
        row = i * blk + lax.broadcasted_iota(jnp.int32, (blk, 1), 0)
        out_ref[:, :] = jnp.where(row < n_valid, o, 0.0)

    return pl.pallas_call(
        body,
        grid=(n_pad // blk,),
        in_specs=[
            pl.BlockSpec((NC, blk, h1), lambda i: (0, i, 0)),
            pl.BlockSpec((blk, h1), lambda i: (i, 0)),
            pl.BlockSpec((blk,), lambda i: (i,)),
            pl.BlockSpec((1, h1), lambda i: (0, 0)),
            pl.BlockSpec((h1, h2), lambda i: (0, 0)),
        ],
        out_specs=pl.BlockSpec((blk, h2), lambda i: (i, 0)),
        out_shape=jax.ShapeDtypeStruct((n_pad, h2), jnp.float32),
    )(aggp, hp1, dinv, b1, W2)


def _tc_head(aggp, hp2, dinv, b2, Wf, bf, n_valid, n_pad, blk):
    h2 = hp2.shape[1]
    c_dim = Wf.shape[1]

    def body(aggp_ref, hp_ref, dinv_ref, b_ref, wf_ref, bf_ref, out_ref):
        i = pl.program_id(0)
        agg = aggp_ref[0, :, :] + aggp_ref[1, :, :]
        dinv = dinv_ref[:]
        t = (agg + hp_ref[:, :]) * dinv[:, None] + b_ref[0, :]
        t = jnp.maximum(t, 0.0)
        row = i * blk + lax.broadcasted_iota(jnp.int32, (blk, 1), 0)
        t = jnp.where(row < n_valid, t, 0.0)
        logits = jnp.dot(t, wf_ref[:, :],
                         preferred_element_type=jnp.float32) + bf_ref[0, :]
        m = jnp.max(logits, axis=1, keepdims=True)
        lse = jnp.log(jnp.sum(jnp.exp(logits - m), axis=1, keepdims=True)) + m
        out_ref[:, :] = logits - lse

    return pl.pallas_call(
        body,
        grid=(n_pad // blk,),
        in_specs=[
            pl.BlockSpec((NC, blk, h2), lambda i: (0, i, 0)),
            pl.BlockSpec((blk, h2), lambda i: (i, 0)),
            pl.BlockSpec((blk,), lambda i: (i,)),
            pl.BlockSpec((1, h2), lambda i: (0, 0)),
            pl.BlockSpec((h2, c_dim), lambda i: (0, 0)),
            pl.BlockSpec((1, c_dim), lambda i: (0, 0)),
        ],
        out_specs=pl.BlockSpec((blk, c_dim), lambda i: (i, 0)),
        out_shape=jax.ShapeDtypeStruct((n_pad, c_dim), jnp.float32),
    )(aggp, hp2, dinv, b2, Wf, bf)


def kernel(feature, edge_index, W1, b1, W2, b2, Wf, bf):
    n, _ = feature.shape
    e = edge_index.shape[1]
    blk = 1024
    n_pad = -(-n // blk) * blk
    epw = NW * K
    e_pad = -(-e // epw) * epw
    cpt = e_pad // epw

    pad_node = n_pad - 1
    src = jnp.concatenate(
        [edge_index[0], jnp.full((e_pad - e,), pad_node, jnp.int32)])
    dst = jnp.concatenate(
        [edge_index[1], jnp.full((e_pad - e,), pad_node, jnp.int32)])
    src_r = src.reshape(NW, cpt, K)
    dst_r = dst.reshape(NW, cpt, K)
    featp = jnp.pad(feature, ((0, n_pad - n), (0, 0)))

    degp = _sc_degree(dst_r, n_pad, cpt)
    hp1, dinv = _tc_pre(featp, W1, degp, n_pad, blk)
    aggp1 = _sc_agg(hp1, src_r, dst_r, n_pad, cpt, W1.shape[1])
    hp2 = _tc_mid(aggp1, hp1, dinv, b1.reshape(1, -1), W2, n, n_pad, blk)
    aggp2 = _sc_agg(hp2, src_r, dst_r, n_pad, cpt, W2.shape[1])
    logp = _tc_head(aggp2, hp2, dinv, b2.reshape(1, -1), Wf,
                    bf.reshape(1, -1), n, n_pad, blk)
    return logp[:n]


# speedup vs baseline: 42.2985x; 1.3850x over previous
import functools

import jax
import jax.numpy as jnp
from jax import lax
from jax.experimental import pallas as pl
from jax.experimental.pallas import tpu as pltpu
from jax.experimental.pallas import tpu_sc as plsc

NC = 2
NS = 16
NW = NC * NS
K = 128
NBUF = 8
SD = 4


def _mesh():
    return plsc.VectorSubcoreMesh(core_axis_name="c", subcore_axis_name="s")


def _sc_degree(dst_r, n_pad, cpt):
    rpt = n_pad // NS

    @functools.partial(
        pl.kernel,
        out_type=jax.ShapeDtypeStruct((NC, n_pad), jnp.float32),
        mesh=_mesh(),
        scratch_types=[
            pltpu.VMEM((cpt, K), jnp.int32),
            pltpu.VMEM((K,), jnp.float32),
            pltpu.VMEM((rpt,), jnp.float32),
            pltpu.VMEM_SHARED((n_pad,), jnp.float32),
        ],
    )
    def k(dst_hbm, out_hbm, dst_v, ones_v, zeros_v, acc_sh):
        c = lax.axis_index("c")
        s = lax.axis_index("s")
        wid = c * NS + s

        def fill_ones(i, _):
            ones_v[pl.ds(i * 16, 16)] = jnp.full((16,), 1.0, jnp.float32)
            return 0

        lax.fori_loop(0, K // 16, fill_ones, 0)

        def fill_zeros(i, _):
            zeros_v[pl.ds(i * 16, 16)] = jnp.zeros((16,), jnp.float32)
            return 0

        lax.fori_loop(0, rpt // 16, fill_zeros, 0)

        pltpu.sync_copy(zeros_v, acc_sh.at[pl.ds(s * rpt, rpt)])
        plsc.subcore_barrier()

        pltpu.sync_copy(dst_hbm.at[wid], dst_v)

        def chunk(j, _):
            pltpu.sync_copy(ones_v, acc_sh.at[dst_v.at[j]], add=True)
            return 0

        lax.fori_loop(0, cpt, chunk, 0)
        plsc.subcore_barrier()
        pltpu.sync_copy(acc_sh.at[pl.ds(s * rpt, rpt)],
                        out_hbm.at[c, pl.ds(s * rpt, rpt)])

    return k(dst_r)


def _sc_agg(hp, src_r, dst_r, n_pad, cpt, h, nbuf, sd):
    rpt = n_pad // NS
    zr = 64

    @functools.partial(
        pl.kernel,
        out_type=jax.ShapeDtypeStruct((NC, n_pad, h), jnp.float32),
        mesh=_mesh(),
        scratch_types=[
            pltpu.VMEM((cpt, K), jnp.int32),
            pltpu.VMEM((cpt, K), jnp.int32),
            pltpu.VMEM((nbuf, K, h), jnp.float32),
            pltpu.VMEM((zr, h), jnp.float32),
            pltpu.SemaphoreType.DMA,
            pltpu.SemaphoreType.DMA,
            pltpu.VMEM_SHARED((n_pad, h), jnp.float32),
            pltpu.VMEM_SHARED((n_pad, h), jnp.float32),
        ],
        compiler_params=pltpu.CompilerParams(use_tc_tiling_on_sc=False),
    )
    def k(hp_hbm, src_hbm, dst_hbm, out_hbm,
          src_v, dst_v, rows_v, zer_v, sem_g, sem_s, acc_sh, hp_sh):
        c = lax.axis_index("c")
        s = lax.axis_index("s")
        wid = c * NS + s

        hvecs = h // 16

        def zf(i, _):
            zer_v[i // hvecs, pl.ds((i % hvecs) * 16, 16)] = (
                jnp.zeros((16,), jnp.float32))
            return 0

        lax.fori_loop(0, zr * hvecs, zf, 0)

        def zc(t, _):
            pltpu.sync_copy(zer_v, acc_sh.at[pl.ds(s * rpt + t * zr, zr)])
            return 0

        lax.fori_loop(0, rpt // zr, zc, 0)
        pltpu.sync_copy(hp_hbm.at[pl.ds(s * rpt, rpt)],
                        hp_sh.at[pl.ds(s * rpt, rpt)])
        plsc.subcore_barrier()

        pltpu.sync_copy(src_hbm.at[wid], src_v)
        pltpu.sync_copy(dst_hbm.at[wid], dst_v)

        for b in range(nbuf - sd):
            pltpu.async_copy(hp_sh.at[src_v.at[b]], rows_v.at[b], sem_g)

        def chunk(j, _):
            @pl.when(j >= sd)
            def _():
                pltpu.make_async_copy(
                    rows_v.at[0], acc_sh.at[dst_v.at[0]], sem_s).wait()

            g = j + nbuf - sd

            @pl.when(g < cpt)
            def _():
                pltpu.async_copy(
                    hp_sh.at[src_v.at[g]], rows_v.at[lax.rem(g, nbuf)],
                    sem_g)

            buf = lax.rem(j, nbuf)
            pltpu.make_async_copy(
                hp_sh.at[src_v.at[j]], rows_v.at[buf], sem_g).wait()
            pltpu.async_copy(
                rows_v.at[buf], acc_sh.at[dst_v.at[j]], sem_s, add=True)
            return 0

        lax.fori_loop(0, cpt, chunk, 0)
        for _ in range(sd):
            pltpu.make_async_copy(
                rows_v.at[0], acc_sh.at[dst_v.at[0]], sem_s).wait()
        plsc.subcore_barrier()
        pltpu.sync_copy(acc_sh.at[pl.ds(s * rpt, rpt)],
                        out_hbm.at[c, pl.ds(s * rpt, rpt)])

    return k(hp, src_r, dst_r)


def _tc_pre(featp, W1, degp, n_pad, blk):
    f = featp.shape[1]
    h1 = W1.shape[1]

    def body(feat_ref, w_ref, degp_ref, hp_ref, dinv_ref):
        deg = degp_ref[0, :] + degp_ref[1, :] + 1.0
        dinv = lax.rsqrt(deg)
        dinv_ref[:] = dinv
        hm = jnp.dot(feat_ref[:, :], w_ref[:, :],
                     preferred_element_type=jnp.float32)
        hp_ref[:, :] = hm * dinv[:, None]

    return pl.pallas_call(
        body,
        grid=(n_pad // blk,),
        in_specs=[
            pl.BlockSpec((blk, f), lambda i: (i, 0)),
            pl.BlockSpec((f, h1), lambda i: (0, 0)),
            pl.BlockSpec((NC, blk), lambda i: (0, i)),
        ],
        out_specs=[
            pl.BlockSpec((blk, h1), lambda i: (i, 0)),
            pl.BlockSpec((blk,), lambda i: (i,)),
        ],
        out_shape=[
            jax.ShapeDtypeStruct((n_pad, h1), jnp.float32),
            jax.ShapeDtypeStruct((n_pad,), jnp.float32),
        ],
    )(featp, W1, degp)


def _tc_mid(aggp, hp1, dinv, b1, W2, n_valid, n_pad, blk):
    h1 = hp1.shape[1]
    h2 = W2.shape[1]

    def body(aggp_ref, hp_ref, dinv_ref, b_ref, w_ref, out_ref):
        i = pl.program_id(0)
        agg = aggp_ref[0, :, :] + aggp_ref[1, :, :]
        dinv = dinv_ref[:]
        t = (agg + hp_ref[:, :]) * dinv[:, None] + b_ref[0, :]
        t = jnp.maximum(t, 0.0)
        o = jnp.dot(t, w_ref[:, :], preferred_element_type=jnp.float32)
        o = o * dinv[:, None]
        row = i * blk + lax.broadcasted_iota(jnp.int32, (blk, 1), 0)
        out_ref[:, :] = jnp.where(row < n_valid, o, 0.0)

    return pl.pallas_call(
        body,
        grid=(n_pad // blk,),
        in_specs=[
            pl.BlockSpec((NC, blk, h1), lambda i: (0, i, 0)),
            pl.BlockSpec((blk, h1), lambda i: (i, 0)),
            pl.BlockSpec((blk,), lambda i: (i,)),
            pl.BlockSpec((1, h1), lambda i: (0, 0)),
            pl.BlockSpec((h1, h2), lambda i: (0, 0)),
        ],
        out_specs=pl.BlockSpec((blk, h2), lambda i: (i, 0)),
        out_shape=jax.ShapeDtypeStruct((n_pad, h2), jnp.float32),
    )(aggp, hp1, dinv, b1, W2)


def _tc_head(aggp, hp2, dinv, b2, Wf, bf, n_valid, n_pad, blk):
    h2 = hp2.shape[1]
    c_dim = Wf.shape[1]

    def body(aggp_ref, hp_ref, dinv_ref, b_ref, wf_ref, bf_ref, out_ref):
        i = pl.program_id(0)
        agg = aggp_ref[0, :, :] + aggp_ref[1, :, :]
        dinv = dinv_ref[:]
        t = (agg + hp_ref[:, :]) * dinv[:, None] + b_ref[0, :]
        t = jnp.maximum(t, 0.0)
        row = i * blk + lax.broadcasted_iota(jnp.int32, (blk, 1), 0)
        t = jnp.where(row < n_valid, t, 0.0)
        logits = jnp.dot(t, wf_ref[:, :],
                         preferred_element_type=jnp.float32) + bf_ref[0, :]
        m = jnp.max(logits, axis=1, keepdims=True)
        lse = jnp.log(jnp.sum(jnp.exp(logits - m), axis=1, keepdims=True)) + m
        out_ref[:, :] = logits - lse

    return pl.pallas_call(
        body,
        grid=(n_pad // blk,),
        in_specs=[
            pl.BlockSpec((NC, blk, h2), lambda i: (0, i, 0)),
            pl.BlockSpec((blk, h2), lambda i: (i, 0)),
            pl.BlockSpec((blk,), lambda i: (i,)),
            pl.BlockSpec((1, h2), lambda i: (0, 0)),
            pl.BlockSpec((h2, c_dim), lambda i: (0, 0)),
            pl.BlockSpec((1, c_dim), lambda i: (0, 0)),
        ],
        out_specs=pl.BlockSpec((blk, c_dim), lambda i: (i, 0)),
        out_shape=jax.ShapeDtypeStruct((n_pad, c_dim), jnp.float32),
    )(aggp, hp2, dinv, b2, Wf, bf)


def kernel(feature, edge_index, W1, b1, W2, b2, Wf, bf):
    n, _ = feature.shape
    e = edge_index.shape[1]
    blk = 1024
    n_pad = -(-n // blk) * blk
    epw = NW * K
    e_pad = -(-e // epw) * epw
    cpt = e_pad // epw

    pad_node = n_pad - 1
    src = jnp.concatenate(
        [edge_index[0], jnp.full((e_pad - e,), pad_node, jnp.int32)])
    dst = jnp.concatenate(
        [edge_index[1], jnp.full((e_pad - e,), pad_node, jnp.int32)])
    src_r = src.reshape(NW, cpt, K)
    dst_r = dst.reshape(NW, cpt, K)
    featp = jnp.pad(feature, ((0, n_pad - n), (0, 0)))

    degp = _sc_degree(dst_r, n_pad, cpt)
    hp1, dinv = _tc_pre(featp, W1, degp, n_pad, blk)
    aggp1 = _sc_agg(hp1, src_r, dst_r, n_pad, cpt, W1.shape[1], 3, 1)
    hp2 = _tc_mid(aggp1, hp1, dinv, b1.reshape(1, -1), W2, n, n_pad, blk)
    aggp2 = _sc_agg(hp2, src_r, dst_r, n_pad, cpt, W2.shape[1], 8, 4)
    logp = _tc_head(aggp2, hp2, dinv, b2.reshape(1, -1), Wf,
                    bf.reshape(1, -1), n, n_pad, blk)
    return logp[:n]


# speedup vs baseline: 43.5497x; 1.0296x over previous
import functools

import jax
import jax.numpy as jnp
from jax import lax
from jax.experimental import pallas as pl
from jax.experimental.pallas import tpu as pltpu
from jax.experimental.pallas import tpu_sc as plsc

NC = 2
NS = 16
NW = NC * NS
K = 128
NBUF = 8
SD = 4


def _mesh():
    return plsc.VectorSubcoreMesh(core_axis_name="c", subcore_axis_name="s")


def _sc_degree(dst_r, n_pad, cpt):
    rpt = n_pad // NS

    @functools.partial(
        pl.kernel,
        out_type=jax.ShapeDtypeStruct((NC, n_pad), jnp.float32),
        mesh=_mesh(),
        scratch_types=[
            pltpu.VMEM((cpt, K), jnp.int32),
            pltpu.VMEM((K,), jnp.float32),
            pltpu.VMEM((rpt,), jnp.float32),
            pltpu.VMEM_SHARED((n_pad,), jnp.float32),
        ],
    )
    def k(dst_hbm, out_hbm, dst_v, ones_v, zeros_v, acc_sh):
        c = lax.axis_index("c")
        s = lax.axis_index("s")
        wid = c * NS + s

        def fill_ones(i, _):
            ones_v[pl.ds(i * 16, 16)] = jnp.full((16,), 1.0, jnp.float32)
            return 0

        lax.fori_loop(0, K // 16, fill_ones, 0)

        def fill_zeros(i, _):
            zeros_v[pl.ds(i * 16, 16)] = jnp.zeros((16,), jnp.float32)
            return 0

        lax.fori_loop(0, rpt // 16, fill_zeros, 0)

        pltpu.sync_copy(zeros_v, acc_sh.at[pl.ds(s * rpt, rpt)])
        plsc.subcore_barrier()

        pltpu.sync_copy(dst_hbm.at[wid], dst_v)

        def chunk(j, _):
            pltpu.sync_copy(ones_v, acc_sh.at[dst_v.at[j]], add=True)
            return 0

        lax.fori_loop(0, cpt, chunk, 0)
        plsc.subcore_barrier()
        pltpu.sync_copy(acc_sh.at[pl.ds(s * rpt, rpt)],
                        out_hbm.at[c, pl.ds(s * rpt, rpt)])

    return k(dst_r)


def _sc_agg(hp, src_r, dst_r, n_pad, cpt, h, nbuf, sd):
    rpt = n_pad // NS
    zr = 64

    @functools.partial(
        pl.kernel,
        out_type=jax.ShapeDtypeStruct((NC, n_pad, h), jnp.float32),
        mesh=_mesh(),
        scratch_types=[
            pltpu.VMEM((cpt, K), jnp.int32),
            pltpu.VMEM((cpt, K), jnp.int32),
            pltpu.VMEM((nbuf, K, h), jnp.float32),
            pltpu.VMEM((zr, h), jnp.float32),
            pltpu.SemaphoreType.DMA,
            pltpu.SemaphoreType.DMA,
            pltpu.VMEM_SHARED((n_pad, h), jnp.float32),
            pltpu.VMEM_SHARED((n_pad, h), jnp.float32),
        ],
        compiler_params=pltpu.CompilerParams(use_tc_tiling_on_sc=False),
    )
    def k(hp_hbm, src_hbm, dst_hbm, out_hbm,
          src_v, dst_v, rows_v, zer_v, sem_g, sem_s, acc_sh, hp_sh):
        c = lax.axis_index("c")
        s = lax.axis_index("s")
        wid = c * NS + s

        hvecs = h // 16

        def zf(i, _):
            zer_v[i // hvecs, pl.ds((i % hvecs) * 16, 16)] = (
                jnp.zeros((16,), jnp.float32))
            return 0

        lax.fori_loop(0, zr * hvecs, zf, 0)

        def zc(t, _):
            pltpu.sync_copy(zer_v, acc_sh.at[pl.ds(s * rpt + t * zr, zr)])
            return 0

        lax.fori_loop(0, rpt // zr, zc, 0)
        pltpu.sync_copy(hp_hbm.at[pl.ds(s * rpt, rpt)],
                        hp_sh.at[pl.ds(s * rpt, rpt)])
        plsc.subcore_barrier()

        pltpu.sync_copy(src_hbm.at[wid], src_v)
        pltpu.sync_copy(dst_hbm.at[wid], dst_v)

        for b in range(nbuf - sd):
            pltpu.async_copy(hp_sh.at[src_v.at[b]], rows_v.at[b], sem_g)

        def chunk(j, _):
            @pl.when(j >= sd)
            def _():
                pltpu.make_async_copy(
                    rows_v.at[0], acc_sh.at[dst_v.at[0]], sem_s).wait()

            g = j + nbuf - sd

            @pl.when(g < cpt)
            def _():
                pltpu.async_copy(
                    hp_sh.at[src_v.at[g]], rows_v.at[lax.rem(g, nbuf)],
                    sem_g)

            buf = lax.rem(j, nbuf)
            pltpu.make_async_copy(
                hp_sh.at[src_v.at[j]], rows_v.at[buf], sem_g).wait()
            pltpu.async_copy(
                rows_v.at[buf], acc_sh.at[dst_v.at[j]], sem_s, add=True)
            return 0

        lax.fori_loop(0, cpt, chunk, 0)
        for _ in range(sd):
            pltpu.make_async_copy(
                rows_v.at[0], acc_sh.at[dst_v.at[0]], sem_s).wait()
        plsc.subcore_barrier()
        pltpu.sync_copy(acc_sh.at[pl.ds(s * rpt, rpt)],
                        out_hbm.at[c, pl.ds(s * rpt, rpt)])

    return k(hp, src_r, dst_r)


def _tc_pre(featp, W1, degp, n_pad, blk):
    f = featp.shape[1]
    h1 = W1.shape[1]

    def body(feat_ref, w_ref, degp_ref, hp_ref, dinv_ref):
        deg = degp_ref[0, :] + degp_ref[1, :] + 1.0
        dinv = lax.rsqrt(deg)
        dinv_ref[:] = dinv
        hm = jnp.dot(feat_ref[:, :], w_ref[:, :],
                     preferred_element_type=jnp.float32)
        hp_ref[:, :] = hm * dinv[:, None]

    return pl.pallas_call(
        body,
        grid=(n_pad // blk,),
        in_specs=[
            pl.BlockSpec((blk, f), lambda i: (i, 0)),
            pl.BlockSpec((f, h1), lambda i: (0, 0)),
            pl.BlockSpec((NC, blk), lambda i: (0, i)),
        ],
        out_specs=[
            pl.BlockSpec((blk, h1), lambda i: (i, 0)),
            pl.BlockSpec((blk,), lambda i: (i,)),
        ],
        out_shape=[
            jax.ShapeDtypeStruct((n_pad, h1), jnp.float32),
            jax.ShapeDtypeStruct((n_pad,), jnp.float32),
        ],
    )(featp, W1, degp)


def _tc_mid(aggp, hp1, dinv, b1, W2, n_valid, n_pad, blk):
    h1 = hp1.shape[1]
    h2 = W2.shape[1]

    def body(aggp_ref, hp_ref, dinv_ref, b_ref, w_ref, out_ref):
        i = pl.program_id(0)
        agg = aggp_ref[0, :, :] + aggp_ref[1, :, :]
        dinv = dinv_ref[:]
        t = (agg + hp_ref[:, :]) * dinv[:, None] + b_ref[0, :]
        t = jnp.maximum(t, 0.0)
        o = jnp.dot(t, w_ref[:, :], preferred_element_type=jnp.float32)
        o = o * dinv[:, None]
        row = i * blk + lax.broadcasted_iota(jnp.int32, (blk, 1), 0)
        out_ref[:, :] = jnp.where(row < n_valid, o, 0.0)

    return pl.pallas_call(
        body,
        grid=(n_pad // blk,),
        in_specs=[
            pl.BlockSpec((NC, blk, h1), lambda i: (0, i, 0)),
            pl.BlockSpec((blk, h1), lambda i: (i, 0)),
            pl.BlockSpec((blk,), lambda i: (i,)),
            pl.BlockSpec((1, h1), lambda i: (0, 0)),
            pl.BlockSpec((h1, h2), lambda i: (0, 0)),
        ],
        out_specs=pl.BlockSpec((blk, h2), lambda i: (i, 0)),
        out_shape=jax.ShapeDtypeStruct((n_pad, h2), jnp.float32),
    )(aggp, hp1, dinv, b1, W2)


def _tc_head(aggp, hp2, dinv, b2, Wf, bf, n_valid, n_pad, blk):
    h2 = hp2.shape[1]
    c_dim = Wf.shape[1]

    def body(aggp_ref, hp_ref, dinv_ref, b_ref, wf_ref, bf_ref, out_ref):
        i = pl.program_id(0)
        agg = aggp_ref[0, :, :] + aggp_ref[1, :, :]
        dinv = dinv_ref[:]
        t = (agg + hp_ref[:, :]) * dinv[:, None] + b_ref[0, :]
        t = jnp.maximum(t, 0.0)
        row = i * blk + lax.broadcasted_iota(jnp.int32, (blk, 1), 0)
        t = jnp.where(row < n_valid, t, 0.0)
        logits = jnp.dot(t, wf_ref[:, :],
                         preferred_element_type=jnp.float32) + bf_ref[0, :]
        m = jnp.max(logits, axis=1, keepdims=True)
        lse = jnp.log(jnp.sum(jnp.exp(logits - m), axis=1, keepdims=True)) + m
        out_ref[:, :] = logits - lse

    return pl.pallas_call(
        body,
        grid=(n_pad // blk,),
        in_specs=[
            pl.BlockSpec((NC, blk, h2), lambda i: (0, i, 0)),
            pl.BlockSpec((blk, h2), lambda i: (i, 0)),
            pl.BlockSpec((blk,), lambda i: (i,)),
            pl.BlockSpec((1, h2), lambda i: (0, 0)),
            pl.BlockSpec((h2, c_dim), lambda i: (0, 0)),
            pl.BlockSpec((1, c_dim), lambda i: (0, 0)),
        ],
        out_specs=pl.BlockSpec((blk, c_dim), lambda i: (i, 0)),
        out_shape=jax.ShapeDtypeStruct((n_pad, c_dim), jnp.float32),
    )(aggp, hp2, dinv, b2, Wf, bf)


def kernel(feature, edge_index, W1, b1, W2, b2, Wf, bf):
    n, _ = feature.shape
    e = edge_index.shape[1]
    blk = 2048
    n_pad = -(-n // blk) * blk
    epw = NW * K
    e_pad = -(-e // epw) * epw
    cpt = e_pad // epw

    pad_node = n_pad - 1
    src = jnp.concatenate(
        [edge_index[0], jnp.full((e_pad - e,), pad_node, jnp.int32)])
    dst = jnp.concatenate(
        [edge_index[1], jnp.full((e_pad - e,), pad_node, jnp.int32)])
    src_r = src.reshape(NW, cpt, K)
    dst_r = dst.reshape(NW, cpt, K)
    featp = jnp.pad(feature, ((0, n_pad - n), (0, 0)))

    degp = _sc_degree(dst_r, n_pad, cpt)
    hp1, dinv = _tc_pre(featp, W1, degp, n_pad, blk)
    aggp1 = _sc_agg(hp1, src_r, dst_r, n_pad, cpt, W1.shape[1], 3, 1)
    hp2 = _tc_mid(aggp1, hp1, dinv, b1.reshape(1, -1), W2, n, n_pad, blk)
    aggp2 = _sc_agg(hp2, src_r, dst_r, n_pad, cpt, W2.shape[1], 8, 4)
    logp = _tc_head(aggp2, hp2, dinv, b2.reshape(1, -1), Wf,
                    bf.reshape(1, -1), n, n_pad, blk)
    return logp[:n]


# speedup vs baseline: 44.5943x; 1.0240x over previous
import functools

import jax
import jax.numpy as jnp
from jax import lax
from jax.experimental import pallas as pl
from jax.experimental.pallas import tpu as pltpu
from jax.experimental.pallas import tpu_sc as plsc

NC = 2
NS = 16
NW = NC * NS


def _chunk_split(epw):
    for k in range(128, 0, -1):
        if epw % k == 0:
            return epw // k, k, 0
    return None


def _mesh():
    return plsc.VectorSubcoreMesh(core_axis_name="c", subcore_axis_name="s")


def _sc_degree(dst_r, n_pad, cpt, kk):
    rpt = n_pad // NS

    @functools.partial(
        pl.kernel,
        out_type=jax.ShapeDtypeStruct((NC, n_pad), jnp.float32),
        mesh=_mesh(),
        scratch_types=[
            pltpu.VMEM((cpt, kk), jnp.int32),
            pltpu.VMEM((128,), jnp.float32),
            pltpu.VMEM((rpt,), jnp.float32),
            pltpu.VMEM_SHARED((n_pad,), jnp.float32),
        ],
    )
    def k(dst_hbm, out_hbm, dst_v, ones_v, zeros_v, acc_sh):
        c = lax.axis_index("c")
        s = lax.axis_index("s")
        wid = c * NS + s

        def fill_ones(i, _):
            ones_v[pl.ds(i * 16, 16)] = jnp.full((16,), 1.0, jnp.float32)
            return 0

        lax.fori_loop(0, 8, fill_ones, 0)

        def fill_zeros(i, _):
            zeros_v[pl.ds(i * 16, 16)] = jnp.zeros((16,), jnp.float32)
            return 0

        lax.fori_loop(0, rpt // 16, fill_zeros, 0)

        pltpu.sync_copy(zeros_v, acc_sh.at[pl.ds(s * rpt, rpt)])
        plsc.subcore_barrier()

        pltpu.sync_copy(dst_hbm.at[wid], dst_v)

        def chunk(j, _):
            pltpu.sync_copy(ones_v.at[pl.ds(0, kk)],
                            acc_sh.at[dst_v.at[j]], add=True)
            return 0

        lax.fori_loop(0, cpt, chunk, 0)
        plsc.subcore_barrier()
        pltpu.sync_copy(acc_sh.at[pl.ds(s * rpt, rpt)],
                        out_hbm.at[c, pl.ds(s * rpt, rpt)])

    return k(dst_r)


def _sc_agg(hp, src_r, dst_r, n_pad, cpt, kk, h, nbuf, sd):
    rpt = n_pad // NS
    zr = 64

    @functools.partial(
        pl.kernel,
        out_type=jax.ShapeDtypeStruct((NC, n_pad, h), jnp.float32),
        mesh=_mesh(),
        scratch_types=[
            pltpu.VMEM((cpt, kk), jnp.int32),
            pltpu.VMEM((cpt, kk), jnp.int32),
            pltpu.VMEM((nbuf, kk, h), jnp.float32),
            pltpu.VMEM((zr, h), jnp.float32),
            pltpu.SemaphoreType.DMA,
            pltpu.SemaphoreType.DMA,
            pltpu.VMEM_SHARED((n_pad, h), jnp.float32),
            pltpu.VMEM_SHARED((n_pad, h), jnp.float32),
        ],
        compiler_params=pltpu.CompilerParams(use_tc_tiling_on_sc=False),
    )
    def k(hp_hbm, src_hbm, dst_hbm, out_hbm,
          src_v, dst_v, rows_v, zer_v, sem_g, sem_s, acc_sh, hp_sh):
        c = lax.axis_index("c")
        s = lax.axis_index("s")
        wid = c * NS + s

        hvecs = h // 16

        def zf(i, _):
            zer_v[i // hvecs, pl.ds((i % hvecs) * 16, 16)] = (
                jnp.zeros((16,), jnp.float32))
            return 0

        lax.fori_loop(0, zr * hvecs, zf, 0)

        def zc(t, _):
            pltpu.sync_copy(zer_v, acc_sh.at[pl.ds(s * rpt + t * zr, zr)])
            return 0

        lax.fori_loop(0, rpt // zr, zc, 0)
        pltpu.sync_copy(hp_hbm.at[pl.ds(s * rpt, rpt)],
                        hp_sh.at[pl.ds(s * rpt, rpt)])
        plsc.subcore_barrier()

        pltpu.sync_copy(src_hbm.at[wid], src_v)
        pltpu.sync_copy(dst_hbm.at[wid], dst_v)

        for b in range(nbuf - sd):
            pltpu.async_copy(hp_sh.at[src_v.at[b]], rows_v.at[b], sem_g)

        def chunk(j, _):
            @pl.when(j >= sd)
            def _():
                pltpu.make_async_copy(
                    rows_v.at[0], acc_sh.at[dst_v.at[0]], sem_s).wait()

            g = j + nbuf - sd

            @pl.when(g < cpt)
            def _():
                pltpu.async_copy(
                    hp_sh.at[src_v.at[g]], rows_v.at[lax.rem(g, nbuf)],
                    sem_g)

            buf = lax.rem(j, nbuf)
            pltpu.make_async_copy(
                hp_sh.at[src_v.at[j]], rows_v.at[buf], sem_g).wait()
            pltpu.async_copy(
                rows_v.at[buf], acc_sh.at[dst_v.at[j]], sem_s, add=True)
            return 0

        lax.fori_loop(0, cpt, chunk, 0)
        for _ in range(sd):
            pltpu.make_async_copy(
                rows_v.at[0], acc_sh.at[dst_v.at[0]], sem_s).wait()
        plsc.subcore_barrier()
        pltpu.sync_copy(acc_sh.at[pl.ds(s * rpt, rpt)],
                        out_hbm.at[c, pl.ds(s * rpt, rpt)])

    return k(hp, src_r, dst_r)


def _tc_pre(featp, W1, degp, n_pad, blk):
    f = featp.shape[1]
    h1 = W1.shape[1]

    def body(feat_ref, w_ref, degp_ref, hp_ref, dinv_ref):
        deg = degp_ref[0, :] + degp_ref[1, :] + 1.0
        dinv = lax.rsqrt(deg)
        dinv_ref[:] = dinv
        hm = jnp.dot(feat_ref[:, :], w_ref[:, :],
                     preferred_element_type=jnp.float32)
        hp_ref[:, :] = hm * dinv[:, None]

    return pl.pallas_call(
        body,
        grid=(n_pad // blk,),
        in_specs=[
            pl.BlockSpec((blk, f), lambda i: (i, 0)),
            pl.BlockSpec((f, h1), lambda i: (0, 0)),
            pl.BlockSpec((NC, blk), lambda i: (0, i)),
        ],
        out_specs=[
            pl.BlockSpec((blk, h1), lambda i: (i, 0)),
            pl.BlockSpec((blk,), lambda i: (i,)),
        ],
        out_shape=[
            jax.ShapeDtypeStruct((n_pad, h1), jnp.float32),
            jax.ShapeDtypeStruct((n_pad,), jnp.float32),
        ],
    )(featp, W1, degp)


def _tc_mid(aggp, hp1, dinv, b1, W2, n_valid, n_pad, blk):
    h1 = hp1.shape[1]
    h2 = W2.shape[1]

    def body(aggp_ref, hp_ref, dinv_ref, b_ref, w_ref, out_ref):
        i = pl.program_id(0)
        agg = aggp_ref[0, :, :] + aggp_ref[1, :, :]
        dinv = dinv_ref[:]
        t = (agg + hp_ref[:, :]) * dinv[:, None] + b_ref[0, :]
        t = jnp.maximum(t, 0.0)
        o = jnp.dot(t, w_ref[:, :], preferred_element_type=jnp.float32)
        o = o * dinv[:, None]
        row = i * blk + lax.broadcasted_iota(jnp.int32, (blk, 1), 0)
        out_ref[:, :] = jnp.where(row < n_valid, o, 0.0)

    return pl.pallas_call(
        body,
        grid=(n_pad // blk,),
        in_specs=[
            pl.BlockSpec((NC, blk, h1), lambda i: (0, i, 0)),
            pl.BlockSpec((blk, h1), lambda i: (i, 0)),
            pl.BlockSpec((blk,), lambda i: (i,)),
            pl.BlockSpec((1, h1), lambda i: (0, 0)),
            pl.BlockSpec((h1, h2), lambda i: (0, 0)),
        ],
        out_specs=pl.BlockSpec((blk, h2), lambda i: (i, 0)),
        out_shape=jax.ShapeDtypeStruct((n_pad, h2), jnp.float32),
    )(aggp, hp1, dinv, b1, W2)


def _tc_head(aggp, hp2, dinv, b2, Wf, bf, n_valid, n_pad, blk):
    h2 = hp2.shape[1]
    c_dim = Wf.shape[1]

    def body(aggp_ref, hp_ref, dinv_ref, b_ref, wf_ref, bf_ref, out_ref):
        i = pl.program_id(0)
        agg = aggp_ref[0, :, :] + aggp_ref[1, :, :]
        dinv = dinv_ref[:]
        t = (agg + hp_ref[:, :]) * dinv[:, None] + b_ref[0, :]
        t = jnp.maximum(t, 0.0)
        row = i * blk + lax.broadcasted_iota(jnp.int32, (blk, 1), 0)
        t = jnp.where(row < n_valid, t, 0.0)
        logits = jnp.dot(t, wf_ref[:, :],
                         preferred_element_type=jnp.float32) + bf_ref[0, :]
        m = jnp.max(logits, axis=1, keepdims=True)
        lse = jnp.log(jnp.sum(jnp.exp(logits - m), axis=1, keepdims=True)) + m
        out_ref[:, :] = logits - lse

    return pl.pallas_call(
        body,
        grid=(n_pad // blk,),
        in_specs=[
            pl.BlockSpec((NC, blk, h2), lambda i: (0, i, 0)),
            pl.BlockSpec((blk, h2), lambda i: (i, 0)),
            pl.BlockSpec((blk,), lambda i: (i,)),
            pl.BlockSpec((1, h2), lambda i: (0, 0)),
            pl.BlockSpec((h2, c_dim), lambda i: (0, 0)),
            pl.BlockSpec((1, c_dim), lambda i: (0, 0)),
        ],
        out_specs=pl.BlockSpec((blk, c_dim), lambda i: (i, 0)),
        out_shape=jax.ShapeDtypeStruct((n_valid, c_dim), jnp.float32),
    )(aggp, hp2, dinv, b2, Wf, bf)


def kernel(feature, edge_index, W1, b1, W2, b2, Wf, bf):
    n, _ = feature.shape
    e = edge_index.shape[1]
    blk = 2048
    n_pad = -(-n // blk) * blk

    split = _chunk_split(e // NW) if e % NW == 0 else None
    if split is not None and split[1] >= 64:
        cpt, kk = split[0], split[1]
        src = edge_index[0]
        dst = edge_index[1]
    else:
        kk = 128
        epw = NW * kk
        e_pad = -(-e // epw) * epw
        cpt = e_pad // epw
        pad_node = n_pad - 1
        src = jnp.concatenate(
            [edge_index[0], jnp.full((e_pad - e,), pad_node, jnp.int32)])
        dst = jnp.concatenate(
            [edge_index[1], jnp.full((e_pad - e,), pad_node, jnp.int32)])
    src_r = src.reshape(NW, cpt, kk)
    dst_r = dst.reshape(NW, cpt, kk)

    degp = _sc_degree(dst_r, n_pad, cpt, kk)
    hp1, dinv = _tc_pre(feature, W1, degp, n_pad, blk)
    aggp1 = _sc_agg(hp1, src_r, dst_r, n_pad, cpt, kk, W1.shape[1], 3, 1)
    hp2 = _tc_mid(aggp1, hp1, dinv, b1.reshape(1, -1), W2, n, n_pad, blk)
    aggp2 = _sc_agg(hp2, src_r, dst_r, n_pad, cpt, kk, W2.shape[1], 8, 4)
    return _tc_head(aggp2, hp2, dinv, b2.reshape(1, -1), Wf,
                    bf.reshape(1, -1), n, n_pad, blk)
